# baseline (speedup 1.0000x reference)
import functools

import jax
import jax.numpy as jnp
from jax import lax
from jax.experimental import pallas as pl
from jax.experimental.pallas import tpu as pltpu
from jax.experimental.pallas import tpu_sc as plsc

TOPK = 4

_SC_PARAMS = pltpu.CompilerParams(use_tc_tiling_on_sc=False,
                                  needs_layout_passes=False)


def _topk_pack_body(x_ref, o_ref):
    xt = x_ref[...]
    cols, rows = xt.shape
    iota = lax.broadcasted_iota(jnp.int32, (cols, rows), 0)
    xb = lax.bitcast_convert_type(xt, jnp.int32)
    tie = jnp.where(xb >= 0, (cols - 1) - iota, iota)
    key = lax.bitcast_convert_type((xb & ~63) | tie, jnp.float32)
    ms = []
    for _ in range(TOPK):
        m = jnp.max(key, axis=0, keepdims=True)
        key = jnp.where(key == m, -jnp.inf, key)
        ms.append(m)
    m_cat = jnp.concatenate(ms, axis=0)
    mb = lax.bitcast_convert_type(m_cat, jnp.int32)
    low = mb & 63
    lanes = jnp.where(mb >= 0, (cols - 1) - low, low)
    e = jnp.exp(m_cat - ms[0])
    soft = e / jnp.sum(e, axis=0, keepdims=True)
    o_ref[...] = jnp.concatenate([soft, lanes.astype(jnp.float32)], axis=0)


def _topk_pack(logits_t, block_rows=2048):
    c, n = logits_t.shape
    grid = -(-n // block_rows)
    n_pad = grid * block_rows
    return pl.pallas_call(
        _topk_pack_body,
        grid=(grid,),
        in_specs=[pl.BlockSpec((c, block_rows), lambda i: (0, i))],
        out_specs=pl.BlockSpec((2 * TOPK, block_rows), lambda i: (0, i)),
        out_shape=jax.ShapeDtypeStruct((2 * TOPK, n_pad), jnp.float32),
    )(logits_t)


def _make_sc_repack(n_pad, nfields):
    info = plsc.get_sparse_core_info()
    nw = info.num_cores * info.num_subcores
    per_w = n_pad // nw
    mesh = plsc.VectorSubcoreMesh(core_axis_name="c", subcore_axis_name="s")

    @functools.partial(
        pl.kernel,
        out_type=jax.ShapeDtypeStruct((n_pad * nfields,), jnp.float32),
        mesh=mesh,
        compiler_params=_SC_PARAMS,
        scratch_types=[
            pltpu.VMEM((nfields * per_w,), jnp.float32),
            pltpu.VMEM((nfields * per_w,), jnp.float32),
        ],
    )
    def sc_repack(fm_hbm, aos_hbm, buf_in, buf_out):
        wid = lax.axis_index("s") * info.num_cores + lax.axis_index("c")
        g0 = wid * per_w
        for f in range(nfields):
            pltpu.sync_copy(fm_hbm.at[pl.ds(f * n_pad + g0, per_w)],
                            buf_in.at[pl.ds(f * per_w, per_w)])

        lanes = lax.broadcasted_iota(jnp.int32, (16,), 0)
        base_src = (lanes % nfields) * per_w + lanes // nfields

        def body(i, carry):
            for u in range(4):
                src = base_src + jnp.full((16,), i * 8 + u * 2, jnp.int32)
                rec = plsc.load_gather(buf_in, [src])
                buf_out[pl.ds(i * 64 + u * 16, 16)] = rec
            return carry

        lax.fori_loop(0, nfields * per_w // 64, body, 0)
        pltpu.sync_copy(buf_out, aos_hbm.at[pl.ds(g0 * nfields,
                                                  nfields * per_w)])

    return sc_repack


def _blend_body(k_hits, a_ref, b_ref, am_ref):
    at = jnp.clip(a_ref[...].T, 0.0, 0.999)
    rows = at.shape[1]
    trans = jnp.ones((1, rows), jnp.float32)
    bls = []
    for k in range(k_hits):
        ak = at[k:k + 1, :]
        bls.append(trans * ak)
        trans = trans * (1.0 - ak)
    blend = jnp.concatenate(bls, axis=0)
    b_ref[...] = blend
    am_ref[...] = jnp.sum(blend, axis=0, keepdims=True)[None]


def _blend_tc(alpha, block_px=2048):
    p, k_hits = alpha.shape
    grid = p // block_px
    body = functools.partial(_blend_body, k_hits)
    return pl.pallas_call(
        body,
        grid=(grid,),
        in_specs=[pl.BlockSpec((block_px, k_hits), lambda i: (i, 0))],
        out_specs=[
            pl.BlockSpec((k_hits, block_px), lambda i: (0, i)),
            pl.BlockSpec((1, 1, block_px), lambda i: (i, 0, 0)),
        ],
        out_shape=[
            jax.ShapeDtypeStruct((k_hits, p), jnp.float32),
            jax.ShapeDtypeStruct((grid, 1, block_px), jnp.float32),
        ],
    )(alpha)


def _make_sc_reduce(n_pad, total_px, cb_dim, k_hits):
    info = plsc.get_sparse_core_info()
    nc = info.num_cores
    nw = nc * info.num_subcores
    chunk_px = 256
    chunk_slots = chunk_px * k_hits
    px_per_w = total_px // nw
    nchunk = px_per_w // chunk_px
    idx_rows = chunk_slots // 128
    wm_words = chunk_px * cb_dim
    mesh = plsc.VectorSubcoreMesh(core_axis_name="c", subcore_axis_name="s")

    @functools.partial(
        pl.kernel,
        out_type=jax.ShapeDtypeStruct((total_px, 128), jnp.float32),
        mesh=mesh,
        compiler_params=_SC_PARAMS,
        scratch_types=[
            pltpu.VMEM((2, idx_rows, 128), jnp.int32),
            pltpu.VMEM((2, chunk_slots, k_hits), jnp.float32),
            pltpu.VMEM((2, k_hits * chunk_px), jnp.float32),
            pltpu.VMEM((chunk_px, cb_dim), jnp.float32),
            pltpu.SemaphoreType.DMA,
            pltpu.SemaphoreType.DMA,
        ],
    )
    def sc_reduce(aos_hbm, idx_hbm, blt_hbm, out_hbm,
                  idx_v, pk_v, bl_v, wm_v, sem, sem_out):
        wid = lax.axis_index("s") * nc + lax.axis_index("c")
        lanes = lax.broadcasted_iota(jnp.int32, (16,), 0)
        rowpat = lanes // 4
        colpat = lanes % 4
        blpat = rowpat * chunk_px
        zeros16 = jnp.zeros((16,), jnp.float32)
        unroll = 4

        def prefetch(c):
            b = c % 2
            r0 = wid * (idx_rows * nchunk) + c * idx_rows
            p0 = wid * px_per_w + c * chunk_px
            pltpu.sync_copy(idx_hbm.at[pl.ds(r0, idx_rows)], idx_v.at[b])
            cps = []
            for j in range(idx_rows):
                cps.append(pltpu.async_copy(
                    aos_hbm.at[idx_v.at[b, j]],
                    pk_v.at[b, pl.ds(j * 128, 128)], sem))
            for k in range(k_hits):
                cps.append(pltpu.async_copy(
                    blt_hbm.at[pl.ds(k * total_px + p0, chunk_px)],
                    bl_v.at[b, pl.ds(k * chunk_px, chunk_px)], sem))
            return cps

        copies = prefetch(0)
        prev_out = None
        for c in range(nchunk):
            b = c % 2
            p0 = wid * px_per_w + c * chunk_px
            if prev_out is not None:
                prev_out.wait()

            def zero_body(i, zc):
                for t in range(2):
                    for j in range(cb_dim // 16):
                        wm_v[i * 2 + t, pl.ds(j * 16, 16)] = zeros16
                return zc
            lax.fori_loop(0, chunk_px // 2, zero_body, 0)
            for cp in copies:
                cp.wait()
            if c + 1 < nchunk:
                copies = prefetch(c + 1)

            @plsc.parallel_loop(0, chunk_px // unroll)
            def px_body(i):
                for u in range(unroll):
                    p = i * unroll + u
                    prow = jnp.full((16,), p, jnp.int32)
                    sp = jnp.full((16,), p * k_hits, jnp.int32) + rowpat
                    blp = prow + blpat
                    for half in range(2):
                        rows = sp + (half * 4)
                        vals = plsc.load_gather(pk_v.at[b], [rows, colpat])
                        idxf = plsc.load_gather(pk_v.at[b],
                                                [rows, colpat + 4])
                        bl = plsc.load_gather(
                            bl_v.at[b], [blp + (half * 4 * chunk_px)])
                        plsc.addupdate_scatter(
                            wm_v, [prow, idxf.astype(jnp.int32)], vals * bl)

            prev_out = pltpu.async_copy(
                wm_v, out_hbm.at[pl.ds(p0, chunk_px), pl.ds(0, cb_dim)],
                sem_out)
        prev_out.wait()

    return sc_reduce


def _decode_body(cb_dim, w_ref, c_ref, f_ref):
    f_ref[...] = jnp.dot(w_ref[...][:, :cb_dim], c_ref[...],
                         preferred_element_type=jnp.float32)


def _decode_matmul(wm, codebook, block_px=512):
    p = wm.shape[0]
    cb_dim, clip_dims = codebook.shape
    grid = p // block_px
    return pl.pallas_call(
        functools.partial(_decode_body, cb_dim),
        grid=(grid,),
        in_specs=[
            pl.BlockSpec((block_px, 128), lambda i: (i, 0)),
            pl.BlockSpec((cb_dim, clip_dims), lambda i: (0, 0)),
        ],
        out_specs=pl.BlockSpec((block_px, clip_dims), lambda i: (i, 0)),
        out_shape=jax.ShapeDtypeStruct((p, clip_dims), jnp.float32),
    )(wm, codebook)


def kernel(world_to_camera, projection, image_width, image_height,
           pixel_gaussian_idx, pixel_alpha, logits, codebooks):
    n, cb_dim = logits.shape
    bz, h, w, k_hits = pixel_alpha.shape
    clip_dims = codebooks.shape[2]
    p = bz * h * w
    total_slots = p * k_hits
    nfields = 2 * TOPK

    idx2d = pixel_gaussian_idx.reshape(total_slots // 128, 128).astype(jnp.int32)
    alpha = pixel_alpha.reshape(p, k_hits)

    fm = _topk_pack(logits.T)
    n_pad = fm.shape[1]
    aos1d = _make_sc_repack(n_pad, nfields)(fm.reshape(nfields * n_pad))
    blt, alpha_map = _blend_tc(alpha)
    wm2d = _make_sc_reduce(n_pad, p, cb_dim, k_hits)(
        aos1d.reshape(n_pad, nfields), idx2d, blt.reshape(k_hits * p))
    feature = _decode_matmul(wm2d, codebooks[0])
    return (feature.reshape(bz, h, w, clip_dims),
            alpha_map.reshape(bz, h, w, 1))


# speedup vs baseline: 1.0023x; 1.0023x over previous
import functools

import jax
import jax.numpy as jnp
from jax import lax
from jax.experimental import pallas as pl
from jax.experimental.pallas import tpu as pltpu
from jax.experimental.pallas import tpu_sc as plsc

TOPK = 4

_SC_PARAMS = pltpu.CompilerParams(use_tc_tiling_on_sc=False,
                                  needs_layout_passes=False)


def _topk_pack_body(x_ref, o_ref):
    xt = x_ref[...]
    cols, rows = xt.shape
    iota = lax.broadcasted_iota(jnp.int32, (cols, rows), 0)
    xb = lax.bitcast_convert_type(xt, jnp.int32)
    tie = jnp.where(xb >= 0, (cols - 1) - iota, iota)
    key = lax.bitcast_convert_type((xb & ~63) | tie, jnp.float32)
    ms = []
    for _ in range(TOPK):
        m = jnp.max(key, axis=0, keepdims=True)
        key = jnp.where(key == m, -jnp.inf, key)
        ms.append(m)
    m_cat = jnp.concatenate(ms, axis=0)
    mb = lax.bitcast_convert_type(m_cat, jnp.int32)
    low = mb & 63
    lanes = jnp.where(mb >= 0, (cols - 1) - low, low)
    e = jnp.exp(m_cat - ms[0])
    soft = e / jnp.sum(e, axis=0, keepdims=True)
    o_ref[...] = jnp.concatenate([soft, lanes.astype(jnp.float32)], axis=0)


def _topk_pack(logits_t, block_rows=2048):
    c, n = logits_t.shape
    grid = -(-n // block_rows)
    n_pad = grid * block_rows
    return pl.pallas_call(
        _topk_pack_body,
        grid=(grid,),
        in_specs=[pl.BlockSpec((c, block_rows), lambda i: (0, i))],
        out_specs=pl.BlockSpec((2 * TOPK, block_rows), lambda i: (0, i)),
        out_shape=jax.ShapeDtypeStruct((2 * TOPK, n_pad), jnp.float32),
    )(logits_t)


def _make_sc_repack(n_pad, nfields):
    info = plsc.get_sparse_core_info()
    nw = info.num_cores * info.num_subcores
    per_w = n_pad // nw
    mesh = plsc.VectorSubcoreMesh(core_axis_name="c", subcore_axis_name="s")

    @functools.partial(
        pl.kernel,
        out_type=jax.ShapeDtypeStruct((n_pad * nfields,), jnp.float32),
        mesh=mesh,
        compiler_params=_SC_PARAMS,
        scratch_types=[
            pltpu.VMEM((nfields * per_w,), jnp.float32),
            pltpu.VMEM((nfields * per_w,), jnp.float32),
        ],
    )
    def sc_repack(fm_hbm, aos_hbm, buf_in, buf_out):
        wid = lax.axis_index("s") * info.num_cores + lax.axis_index("c")
        g0 = wid * per_w
        for f in range(nfields):
            pltpu.sync_copy(fm_hbm.at[pl.ds(f * n_pad + g0, per_w)],
                            buf_in.at[pl.ds(f * per_w, per_w)])

        lanes = lax.broadcasted_iota(jnp.int32, (16,), 0)
        base_src = (lanes % nfields) * per_w + lanes // nfields

        @plsc.parallel_loop(0, nfields * per_w // 64)
        def body(i):
            for u in range(4):
                src = base_src + jnp.full((16,), i * 8 + u * 2, jnp.int32)
                rec = plsc.load_gather(buf_in, [src])
                buf_out[pl.ds(i * 64 + u * 16, 16)] = rec
        pltpu.sync_copy(buf_out, aos_hbm.at[pl.ds(g0 * nfields,
                                                  nfields * per_w)])

    return sc_repack


def _blend_body(k_hits, a_ref, b_ref, am_ref):
    at = jnp.clip(a_ref[...].T, 0.0, 0.999)
    rows = at.shape[1]
    trans = jnp.ones((1, rows), jnp.float32)
    bls = []
    for k in range(k_hits):
        ak = at[k:k + 1, :]
        bls.append(trans * ak)
        trans = trans * (1.0 - ak)
    blend = jnp.concatenate(bls, axis=0)
    b_ref[...] = blend
    am_ref[...] = jnp.sum(blend, axis=0, keepdims=True)[None]


def _blend_tc(alpha, block_px=2048):
    p, k_hits = alpha.shape
    grid = p // block_px
    body = functools.partial(_blend_body, k_hits)
    return pl.pallas_call(
        body,
        grid=(grid,),
        in_specs=[pl.BlockSpec((block_px, k_hits), lambda i: (i, 0))],
        out_specs=[
            pl.BlockSpec((k_hits, block_px), lambda i: (0, i)),
            pl.BlockSpec((1, 1, block_px), lambda i: (i, 0, 0)),
        ],
        out_shape=[
            jax.ShapeDtypeStruct((k_hits, p), jnp.float32),
            jax.ShapeDtypeStruct((grid, 1, block_px), jnp.float32),
        ],
    )(alpha)


def _make_sc_reduce(n_pad, total_px, cb_dim, k_hits):
    info = plsc.get_sparse_core_info()
    nc = info.num_cores
    nw = nc * info.num_subcores
    chunk_px = 256
    chunk_slots = chunk_px * k_hits
    px_per_w = total_px // nw
    nchunk = px_per_w // chunk_px
    idx_rows = chunk_slots // 128
    wm_words = chunk_px * cb_dim
    mesh = plsc.VectorSubcoreMesh(core_axis_name="c", subcore_axis_name="s")

    @functools.partial(
        pl.kernel,
        out_type=jax.ShapeDtypeStruct((total_px, 128), jnp.float32),
        mesh=mesh,
        compiler_params=_SC_PARAMS,
        scratch_types=[
            pltpu.VMEM((2, idx_rows, 128), jnp.int32),
            pltpu.VMEM((2, chunk_slots, k_hits), jnp.float32),
            pltpu.VMEM((2, k_hits * chunk_px), jnp.float32),
            pltpu.VMEM((chunk_px, cb_dim), jnp.float32),
            pltpu.SemaphoreType.DMA,
            pltpu.SemaphoreType.DMA,
        ],
    )
    def sc_reduce(aos_hbm, idx_hbm, blt_hbm, out_hbm,
                  idx_v, pk_v, bl_v, wm_v, sem, sem_out):
        wid = lax.axis_index("s") * nc + lax.axis_index("c")
        lanes = lax.broadcasted_iota(jnp.int32, (16,), 0)
        rowpat = lanes // 4
        colpat = lanes % 4
        blpat = rowpat * chunk_px
        zeros16 = jnp.zeros((16,), jnp.float32)
        unroll = 4

        def prefetch(c):
            b = c % 2
            r0 = wid * (idx_rows * nchunk) + c * idx_rows
            p0 = wid * px_per_w + c * chunk_px
            pltpu.sync_copy(idx_hbm.at[pl.ds(r0, idx_rows)], idx_v.at[b])
            cps = []
            for j in range(idx_rows):
                cps.append(pltpu.async_copy(
                    aos_hbm.at[idx_v.at[b, j]],
                    pk_v.at[b, pl.ds(j * 128, 128)], sem))
            for k in range(k_hits):
                cps.append(pltpu.async_copy(
                    blt_hbm.at[pl.ds(k * total_px + p0, chunk_px)],
                    bl_v.at[b, pl.ds(k * chunk_px, chunk_px)], sem))
            return cps

        copies = prefetch(0)
        prev_out = None
        for c in range(nchunk):
            b = c % 2
            p0 = wid * px_per_w + c * chunk_px
            if prev_out is not None:
                prev_out.wait()

            @plsc.parallel_loop(0, chunk_px // 2)
            def zero_body(i):
                for t in range(2):
                    for j in range(cb_dim // 16):
                        wm_v[i * 2 + t, pl.ds(j * 16, 16)] = zeros16
            for cp in copies:
                cp.wait()
            if c + 1 < nchunk:
                copies = prefetch(c + 1)

            @plsc.parallel_loop(0, chunk_px // unroll)
            def px_body(i):
                for u in range(unroll):
                    p = i * unroll + u
                    prow = jnp.full((16,), p, jnp.int32)
                    sp = jnp.full((16,), p * k_hits, jnp.int32) + rowpat
                    blp = prow + blpat
                    for half in range(2):
                        rows = sp + (half * 4)
                        vals = plsc.load_gather(pk_v.at[b], [rows, colpat])
                        idxf = plsc.load_gather(pk_v.at[b],
                                                [rows, colpat + 4])
                        bl = plsc.load_gather(
                            bl_v.at[b], [blp + (half * 4 * chunk_px)])
                        plsc.addupdate_scatter(
                            wm_v, [prow, idxf.astype(jnp.int32)], vals * bl)

            prev_out = pltpu.async_copy(
                wm_v, out_hbm.at[pl.ds(p0, chunk_px), pl.ds(0, cb_dim)],
                sem_out)
        prev_out.wait()

    return sc_reduce


def _decode_body(cb_dim, w_ref, c_ref, f_ref):
    f_ref[...] = jnp.dot(w_ref[...][:, :cb_dim], c_ref[...],
                         preferred_element_type=jnp.float32)


def _decode_matmul(wm, codebook, block_px=512):
    p = wm.shape[0]
    cb_dim, clip_dims = codebook.shape
    grid = p // block_px
    return pl.pallas_call(
        functools.partial(_decode_body, cb_dim),
        grid=(grid,),
        in_specs=[
            pl.BlockSpec((block_px, 128), lambda i: (i, 0)),
            pl.BlockSpec((cb_dim, clip_dims), lambda i: (0, 0)),
        ],
        out_specs=pl.BlockSpec((block_px, clip_dims), lambda i: (i, 0)),
        out_shape=jax.ShapeDtypeStruct((p, clip_dims), jnp.float32),
    )(wm, codebook)


def kernel(world_to_camera, projection, image_width, image_height,
           pixel_gaussian_idx, pixel_alpha, logits, codebooks):
    n, cb_dim = logits.shape
    bz, h, w, k_hits = pixel_alpha.shape
    clip_dims = codebooks.shape[2]
    p = bz * h * w
    total_slots = p * k_hits
    nfields = 2 * TOPK

    idx2d = pixel_gaussian_idx.reshape(total_slots // 128, 128).astype(jnp.int32)
    alpha = pixel_alpha.reshape(p, k_hits)

    fm = _topk_pack(logits.T)
    n_pad = fm.shape[1]
    aos1d = _make_sc_repack(n_pad, nfields)(fm.reshape(nfields * n_pad))
    blt, alpha_map = _blend_tc(alpha)
    wm2d = _make_sc_reduce(n_pad, p, cb_dim, k_hits)(
        aos1d.reshape(n_pad, nfields), idx2d, blt.reshape(k_hits * p))
    feature = _decode_matmul(wm2d, codebooks[0])
    return (feature.reshape(bz, h, w, clip_dims),
            alpha_map.reshape(bz, h, w, 1))
